# baseline (speedup 1.0000x reference)
import functools

import jax
import jax.numpy as jnp
from jax import lax
from jax.experimental import pallas as pl
from jax.experimental.pallas import tpu as pltpu
from jax.experimental.pallas import tpu_sc as plsc

_LANES = 16


def kernel(input_ids, word_embeddings, position_embeddings):
    batch, seqlen = input_ids.shape
    _, dim = word_embeddings.shape

    num_cores, num_subcores = 2, 16
    num_workers = num_cores * num_subcores
    pos_per_worker = seqlen // num_workers
    chunk = 32
    num_chunks = pos_per_worker // chunk

    mesh = plsc.VectorSubcoreMesh(core_axis_name="c", subcore_axis_name="s")

    @functools.partial(
        pl.kernel,
        out_type=jax.ShapeDtypeStruct((batch, seqlen, dim), jnp.float32),
        mesh=mesh,
        scratch_types=[
            pltpu.VMEM((chunk,), jnp.int32),
            pltpu.VMEM((chunk, dim), jnp.float32),
            pltpu.VMEM((chunk, dim), jnp.float32),
            pltpu.SemaphoreType.DMA,
        ],
    )
    def emb_kernel(ids_hbm, word_hbm, pos_hbm, out_hbm, idx_v, w_v, p_v, sem):
        wid = lax.axis_index("s") * num_cores + lax.axis_index("c")
        pos_base = wid * pos_per_worker

        @pl.loop(0, num_chunks)
        def _chunk(ci):
            p0 = pos_base + ci * chunk
            pltpu.sync_copy(pos_hbm.at[pl.ds(p0, chunk)], p_v)
            for b in range(batch):
                pltpu.sync_copy(ids_hbm.at[b, pl.ds(p0, chunk)], idx_v)
                pltpu.async_copy(word_hbm.at[idx_v], w_v, sem).wait()

                @pl.loop(0, chunk)
                def _row(r):
                    @pl.loop(0, dim // _LANES)
                    def _vec(j):
                        sl = pl.ds(j * _LANES, _LANES)
                        plsc.addupdate(w_v.at[r, sl], p_v[r, sl])

                pltpu.sync_copy(w_v, out_hbm.at[b, pl.ds(p0, chunk)])

    return emb_kernel(input_ids, word_embeddings, position_embeddings)


# speedup vs baseline: 1.4035x; 1.4035x over previous
import functools

import jax
import jax.numpy as jnp
from jax import lax
from jax.experimental import pallas as pl
from jax.experimental.pallas import tpu as pltpu
from jax.experimental.pallas import tpu_sc as plsc

_LANES = 16


def kernel(input_ids, word_embeddings, position_embeddings):
    batch, seqlen = input_ids.shape
    _, dim = word_embeddings.shape

    num_cores, num_subcores = 2, 16
    num_workers = num_cores * num_subcores
    pos_per_worker = seqlen // num_workers
    chunk = 16
    num_chunks = pos_per_worker // chunk

    mesh = plsc.VectorSubcoreMesh(core_axis_name="c", subcore_axis_name="s")

    @functools.partial(
        pl.kernel,
        out_type=jax.ShapeDtypeStruct((batch, seqlen, dim), jnp.float32),
        mesh=mesh,
        scratch_types=(
            [pltpu.VMEM((batch, pos_per_worker), jnp.int32)]
            + [pltpu.VMEM((chunk, dim), jnp.float32)] * batch
            + [pltpu.VMEM((chunk, dim), jnp.float32)]
            + [pltpu.SemaphoreType.DMA] * (2 * batch)
        ),
    )
    def emb_kernel(ids_hbm, word_hbm, pos_hbm, out_hbm, idx_v, *rest):
        wbuf = rest[:batch]
        p_v = rest[batch]
        gsem = rest[batch + 1:batch + 1 + batch]
        osem = rest[batch + 1 + batch:]

        wid = lax.axis_index("s") * num_cores + lax.axis_index("c")
        pos_base = wid * pos_per_worker

        def gather_desc(ci, b):
            src = word_hbm.at[idx_v.at[b, pl.ds(ci * chunk, chunk)]]
            return pltpu.make_async_copy(src, wbuf[b], gsem[b])

        def out_desc(ci, b):
            dst = out_hbm.at[b, pl.ds(pos_base + ci * chunk, chunk)]
            return pltpu.make_async_copy(wbuf[b], dst, osem[b])

        for b in range(batch):
            pltpu.sync_copy(ids_hbm.at[b, pl.ds(pos_base, pos_per_worker)],
                            idx_v.at[b])
        for b in range(batch):
            gather_desc(0, b).start()

        @pl.loop(0, num_chunks)
        def _chunk(ci):
            @pl.when(ci > 0)
            def _refill():
                for b in range(batch):
                    out_desc(ci, b).wait()
                    gather_desc(ci, b).start()

            pltpu.sync_copy(pos_hbm.at[pl.ds(pos_base + ci * chunk, chunk)],
                            p_v)

            for b in range(batch):
                gather_desc(ci, b).wait()

                @pl.loop(0, chunk)
                def _row(r):
                    for j in range(dim // _LANES):
                        sl = pl.ds(j * _LANES, _LANES)
                        plsc.addupdate(wbuf[b].at[r, sl], p_v[r, sl])

                out_desc(ci, b).start()

        for b in range(batch):
            out_desc(num_chunks - 1, b).wait()

    return emb_kernel(input_ids, word_embeddings, position_embeddings)


# speedup vs baseline: 2.2821x; 1.6259x over previous
import functools

import jax
import jax.numpy as jnp
from jax import lax
from jax.experimental import pallas as pl
from jax.experimental.pallas import tpu as pltpu
from jax.experimental.pallas import tpu_sc as plsc

_LANES = 16


def kernel(input_ids, word_embeddings, position_embeddings):
    batch, seqlen = input_ids.shape
    _, dim = word_embeddings.shape

    num_cores, num_subcores = 2, 16
    num_workers = num_cores * num_subcores
    pos_per_worker = seqlen // num_workers
    chunk = 16
    num_chunks = pos_per_worker // chunk

    mesh = plsc.VectorSubcoreMesh(core_axis_name="c", subcore_axis_name="s")

    @functools.partial(
        pl.kernel,
        out_type=jax.ShapeDtypeStruct((batch, seqlen, dim), jnp.float32),
        mesh=mesh,
        scratch_types=(
            [pltpu.VMEM((batch, pos_per_worker), jnp.int32)]
            + [pltpu.VMEM((chunk, dim), jnp.float32)] * batch
            + [pltpu.VMEM((chunk, dim), jnp.float32)]
            + [pltpu.SemaphoreType.DMA] * (2 * batch + 1)
        ),
    )
    def emb_kernel(ids_hbm, word_hbm, pos_hbm, out_hbm, idx_v, *rest):
        wbuf = rest[:batch]
        p_v = rest[batch]
        gsem = rest[batch + 1:2 * batch + 1]
        osem = rest[2 * batch + 1:3 * batch + 1]
        psem = rest[3 * batch + 1]

        wid = lax.axis_index("s") * num_cores + lax.axis_index("c")
        pos_base = wid * pos_per_worker

        def pos_desc(ci):
            src = pos_hbm.at[pl.ds(pos_base + ci * chunk, chunk)]
            return pltpu.make_async_copy(src, p_v, psem)

        def gather_desc(ci, b):
            src = word_hbm.at[idx_v.at[b, pl.ds(ci * chunk, chunk)]]
            return pltpu.make_async_copy(src, wbuf[b], gsem[b])

        def out_desc(ci, b):
            dst = out_hbm.at[b, pl.ds(pos_base + ci * chunk, chunk)]
            return pltpu.make_async_copy(wbuf[b], dst, osem[b])

        for b in range(batch):
            pltpu.sync_copy(ids_hbm.at[b, pl.ds(pos_base, pos_per_worker)],
                            idx_v.at[b])
        pos_desc(0).start()
        for b in range(batch):
            gather_desc(0, b).start()

        @pl.loop(0, num_chunks)
        def _chunk(ci):
            @pl.when(ci > 0)
            def _refill():
                for b in range(batch):
                    out_desc(ci, b).wait()
                    gather_desc(ci, b).start()

            for b in range(batch):
                gather_desc(ci, b).wait()
            pos_desc(ci).wait()

            @pl.loop(0, chunk)
            def _row(r):
                for j in range(dim // _LANES):
                    sl = pl.ds(j * _LANES, _LANES)
                    x = p_v[r, sl]
                    for b in range(batch):
                        plsc.addupdate(wbuf[b].at[r, sl], x)

            @pl.when(ci < num_chunks - 1)
            def _prefetch_pos():
                pos_desc(ci + 1).start()

            for b in range(batch):
                out_desc(ci, b).start()

        for b in range(batch):
            out_desc(num_chunks - 1, b).wait()

    return emb_kernel(input_ids, word_embeddings, position_embeddings)


# speedup vs baseline: 3.3412x; 1.4641x over previous
import functools

import jax
import jax.numpy as jnp
from jax import lax
from jax.experimental import pallas as pl
from jax.experimental.pallas import tpu as pltpu
from jax.experimental.pallas import tpu_sc as plsc

_LANES = 16
_NSETS = 3


def kernel(input_ids, word_embeddings, position_embeddings):
    batch, seqlen = input_ids.shape
    _, dim = word_embeddings.shape

    num_cores, num_subcores = 2, 16
    num_workers = num_cores * num_subcores
    pos_per_worker = seqlen // num_workers
    chunk = 8
    num_chunks = pos_per_worker // chunk

    mesh = plsc.VectorSubcoreMesh(core_axis_name="c", subcore_axis_name="s")

    @functools.partial(
        pl.kernel,
        out_type=jax.ShapeDtypeStruct((batch, seqlen, dim), jnp.float32),
        mesh=mesh,
        scratch_types=(
            [pltpu.VMEM((batch, pos_per_worker), jnp.int32)]
            + [pltpu.VMEM((chunk, dim), jnp.float32)] * (_NSETS * batch)
            + [pltpu.VMEM((chunk, dim), jnp.float32)] * _NSETS
            + [pltpu.SemaphoreType.DMA] * (_NSETS * batch)
            + [pltpu.SemaphoreType.DMA] * (_NSETS * batch)
            + [pltpu.SemaphoreType.DMA] * _NSETS
        ),
    )
    def emb_kernel(ids_hbm, word_hbm, pos_hbm, out_hbm, idx_v, *rest):
        nw = _NSETS * batch
        wbuf = [rest[s * batch:(s + 1) * batch] for s in range(_NSETS)]
        pbuf = rest[nw:nw + _NSETS]
        base = nw + _NSETS
        gsem = [rest[base + s * batch:base + (s + 1) * batch]
                for s in range(_NSETS)]
        osem = [rest[base + nw + s * batch:base + nw + (s + 1) * batch]
                for s in range(_NSETS)]
        psem = rest[base + 2 * nw:]

        wid = lax.axis_index("s") * num_cores + lax.axis_index("c")
        pos_base = wid * pos_per_worker

        def pos_desc(ci, s):
            src = pos_hbm.at[pl.ds(pos_base + ci * chunk, chunk)]
            return pltpu.make_async_copy(src, pbuf[s], psem[s])

        def gather_desc(ci, s, b):
            src = word_hbm.at[idx_v.at[b, pl.ds(ci * chunk, chunk)]]
            return pltpu.make_async_copy(src, wbuf[s][b], gsem[s][b])

        def out_desc(ci, s, b):
            dst = out_hbm.at[b, pl.ds(pos_base + ci * chunk, chunk)]
            return pltpu.make_async_copy(wbuf[s][b], dst, osem[s][b])

        for b in range(batch):
            pltpu.sync_copy(ids_hbm.at[b, pl.ds(pos_base, pos_per_worker)],
                            idx_v.at[b])
        pos_desc(0, 0).start()
        for b in range(batch):
            gather_desc(0, 0, b).start()

        def step(ci, s):
            snext = (s + 1) % _NSETS

            @pl.when(ci + 1 < num_chunks)
            def _refill():
                @pl.when(ci >= _NSETS - 1)
                def _drain():
                    for b in range(batch):
                        out_desc(ci, snext, b).wait()
                for b in range(batch):
                    gather_desc(ci + 1, snext, b).start()
                pos_desc(ci + 1, snext).start()

            for b in range(batch):
                gather_desc(ci, s, b).wait()
            pos_desc(ci, s).wait()

            @pl.loop(0, chunk)
            def _row(r):
                for j in range(dim // _LANES):
                    sl = pl.ds(j * _LANES, _LANES)
                    x = pbuf[s][r, sl]
                    for b in range(batch):
                        plsc.addupdate(wbuf[s][b].at[r, sl], x)

            for b in range(batch):
                out_desc(ci, s, b).start()

        main = (num_chunks // _NSETS) * _NSETS

        @pl.loop(0, main, step=_NSETS)
        def _chunks(cio):
            for si in range(_NSETS):
                step(cio + si, si)

        for ci in range(main, num_chunks):
            step(ci, ci % _NSETS)

        for ci in range(num_chunks - _NSETS, num_chunks):
            s = ci % _NSETS
            for b in range(batch):
                out_desc(ci, s, b).wait()

    return emb_kernel(input_ids, word_embeddings, position_embeddings)


# speedup vs baseline: 3.4147x; 1.0220x over previous
import functools

import jax
import jax.numpy as jnp
from jax import lax
from jax.experimental import pallas as pl
from jax.experimental.pallas import tpu as pltpu
from jax.experimental.pallas import tpu_sc as plsc

_LANES = 16
_NSETS = 3


def kernel(input_ids, word_embeddings, position_embeddings):
    batch, seqlen = input_ids.shape
    _, dim = word_embeddings.shape

    num_cores, num_subcores = 2, 16
    num_workers = num_cores * num_subcores
    pos_per_worker = seqlen // num_workers
    chunk = 8
    num_chunks = pos_per_worker // chunk
    rows = batch * chunk

    mesh = plsc.VectorSubcoreMesh(core_axis_name="c", subcore_axis_name="s")

    @functools.partial(
        pl.kernel,
        out_type=jax.ShapeDtypeStruct((batch, seqlen, dim), jnp.float32),
        mesh=mesh,
        scratch_types=(
            [pltpu.VMEM((num_chunks * rows,), jnp.int32)]
            + [pltpu.VMEM((rows, dim), jnp.float32)] * _NSETS
            + [pltpu.VMEM((chunk, dim), jnp.float32)] * _NSETS
            + [pltpu.SemaphoreType.DMA] * _NSETS
            + [pltpu.SemaphoreType.DMA] * _NSETS
            + [pltpu.SemaphoreType.DMA] * _NSETS
            + [pltpu.SemaphoreType.DMA]
        ),
    )
    def emb_kernel(ids_hbm, word_hbm, pos_hbm, out_hbm, idx_v, *rest):
        wbuf = rest[0:_NSETS]
        pbuf = rest[_NSETS:2 * _NSETS]
        gsem = rest[2 * _NSETS:3 * _NSETS]
        osem = rest[3 * _NSETS:4 * _NSETS]
        psem = rest[4 * _NSETS:5 * _NSETS]
        isem = rest[5 * _NSETS]

        wid = lax.axis_index("s") * num_cores + lax.axis_index("c")
        pos_base = wid * pos_per_worker

        def pos_desc(ci, s):
            src = pos_hbm.at[pl.ds(pos_base + ci * chunk, chunk)]
            return pltpu.make_async_copy(src, pbuf[s], psem[s])

        def gather_desc(ci, s):
            src = word_hbm.at[idx_v.at[pl.ds(ci * rows, rows)]]
            return pltpu.make_async_copy(src, wbuf[s], gsem[s])

        def out_start(ci, s):
            for b in range(batch):
                dst = out_hbm.at[b, pl.ds(pos_base + ci * chunk, chunk)]
                pltpu.make_async_copy(
                    wbuf[s].at[pl.ds(b * chunk, chunk)], dst, osem[s]).start()

        def out_drain(s):
            dst = out_hbm.at[0, pl.ds(0, rows)]
            pltpu.make_async_copy(wbuf[s], dst, osem[s]).wait()

        for ci in range(num_chunks):
            for b in range(batch):
                pltpu.make_async_copy(
                    ids_hbm.at[b, pl.ds(pos_base + ci * chunk, chunk)],
                    idx_v.at[pl.ds(ci * rows + b * chunk, chunk)],
                    isem).start()
        pltpu.make_async_copy(
            ids_hbm.at[0, pl.ds(0, num_chunks * rows)], idx_v, isem).wait()

        pos_desc(0, 0).start()
        gather_desc(0, 0).start()

        def step(ci, s):
            snext = (s + 1) % _NSETS

            @pl.when(ci + 1 < num_chunks)
            def _refill():
                @pl.when(ci >= _NSETS - 1)
                def _drain():
                    out_drain(snext)
                gather_desc(ci + 1, snext).start()
                pos_desc(ci + 1, snext).start()

            gather_desc(ci, s).wait()
            pos_desc(ci, s).wait()

            @pl.loop(0, chunk)
            def _row(r):
                for j in range(dim // _LANES):
                    sl = pl.ds(j * _LANES, _LANES)
                    x = pbuf[s][r, sl]
                    for b in range(batch):
                        plsc.addupdate(wbuf[s].at[b * chunk + r, sl], x)

            out_start(ci, s)

        main = (num_chunks // _NSETS) * _NSETS

        @pl.loop(0, main, step=_NSETS)
        def _chunks(cio):
            for si in range(_NSETS):
                step(cio + si, si)

        for ci in range(main, num_chunks):
            step(ci, ci % _NSETS)

        for ci in range(num_chunks - _NSETS, num_chunks):
            out_drain(ci % _NSETS)

    return emb_kernel(input_ids, word_embeddings, position_embeddings)
